# initial kernel scaffold (unmeasured)
import jax
import jax.numpy as jnp
from jax import lax
from jax.experimental import pallas as pl
from jax.experimental.pallas import tpu as pltpu


def kernel(
    x,
):
    def body(*refs):
        pass

    out_shape = jax.ShapeDtypeStruct(..., jnp.float32)
    return pl.pallas_call(body, out_shape=out_shape)(...)


# baseline (device time: 203485 ns/iter reference)
import os

import jax
import jax.numpy as jnp
from jax import lax
from jax.experimental import pallas as pl
from jax.experimental.pallas import tpu as pltpu

N_DEV = 8
_INTERPRET = os.environ.get("KERNEL_INTERPRET") == "1"


def kernel(x):
    m, n = x.shape
    assert m % N_DEV == 0
    mc = m // N_DEV

    def body(x_ref, out_ref, send_buf, recv_buf, send_sems, recv_sems):
        my = lax.axis_index("i")
        right = lax.rem(my + 1, N_DEV)

        def chunk(idx):
            return x_ref[pl.ds(idx * mc, mc), :].astype(jnp.bfloat16)

        for k in range(2 * (N_DEV - 1)):
            if k == 0:
                send_buf[0] = chunk(my)
                src = send_buf.at[0]
            elif k < N_DEV - 1:
                sb = k % 2
                send_buf[sb] = recv_buf[(k - 1) % N_DEV] + chunk((my - k) % N_DEV)
                src = send_buf.at[sb]
            elif k == N_DEV - 1:
                red_idx = (my + 1) % N_DEV
                send_buf[1] = recv_buf[(k - 1) % N_DEV] + chunk(red_idx)
                out_ref[pl.ds(red_idx * mc, mc), :] = send_buf[1].astype(
                    out_ref.dtype
                )
                src = send_buf.at[1]
            else:
                src = recv_buf.at[(k - 1) % N_DEV]

            rdma = pltpu.make_async_remote_copy(
                src_ref=src,
                dst_ref=recv_buf.at[k % N_DEV],
                send_sem=send_sems.at[k],
                recv_sem=recv_sems.at[k],
                device_id=(right,),
                device_id_type=pl.DeviceIdType.MESH,
            )
            rdma.start()
            rdma.wait()

            if k >= N_DEV - 1:
                t = k - (N_DEV - 1)
                out_ref[pl.ds(((my - t) % N_DEV) * mc, mc), :] = recv_buf[
                    k % N_DEV
                ].astype(out_ref.dtype)

    return pl.pallas_call(
        body,
        out_shape=jax.ShapeDtypeStruct((m, n), jnp.float32),
        in_specs=[pl.BlockSpec(memory_space=pltpu.VMEM)],
        out_specs=pl.BlockSpec(memory_space=pltpu.VMEM),
        scratch_shapes=[
            pltpu.VMEM((2, mc, n), jnp.bfloat16),
            pltpu.VMEM((N_DEV, mc, n), jnp.bfloat16),
            pltpu.SemaphoreType.DMA((2 * (N_DEV - 1),)),
            pltpu.SemaphoreType.DMA((2 * (N_DEV - 1),)),
        ],
        interpret=pltpu.InterpretParams() if _INTERPRET else False,
    )(x)


# device time: 130318 ns/iter; 1.5614x vs baseline; 1.5614x over previous
import os

import jax
import jax.numpy as jnp
from jax import lax
from jax.experimental import pallas as pl
from jax.experimental.pallas import tpu as pltpu

N_DEV = 8
_INTERPRET = os.environ.get("KERNEL_INTERPRET") == "1"


def kernel(x):
    m, n = x.shape
    assert m % N_DEV == 0 and n % 2 == 0
    mc = m // N_DEV
    nh = n // 2

    def body(x_ref, out_ref, sbuf_f, sbuf_b, rbuf_f, rbuf_b,
             ssem_f, rsem_f, ssem_b, rsem_b):
        my = lax.axis_index("i")
        right = lax.rem(my + 1, N_DEV)
        left = lax.rem(my + N_DEV - 1, N_DEV)

        def chunk_f(idx):
            return x_ref[pl.ds(idx * mc, mc), :nh].astype(jnp.bfloat16)

        def chunk_b(idx):
            return x_ref[pl.ds(idx * mc, mc), nh:].astype(jnp.bfloat16)

        for k in range(2 * (N_DEV - 1)):
            if k == 0:
                sbuf_f[0] = chunk_f(my)
                sbuf_b[0] = chunk_b(my)
                src_f = sbuf_f.at[0]
                src_b = sbuf_b.at[0]
            elif k < N_DEV - 1:
                sb = k % 2
                sbuf_f[sb] = rbuf_f[(k - 1) % N_DEV] + chunk_f((my - k) % N_DEV)
                sbuf_b[sb] = rbuf_b[(k - 1) % N_DEV] + chunk_b((my + k) % N_DEV)
                src_f = sbuf_f.at[sb]
                src_b = sbuf_b.at[sb]
            elif k == N_DEV - 1:
                rf = (my + 1) % N_DEV
                rb = (my + N_DEV - 1) % N_DEV
                sbuf_f[1] = rbuf_f[(k - 1) % N_DEV] + chunk_f(rf)
                sbuf_b[1] = rbuf_b[(k - 1) % N_DEV] + chunk_b(rb)
                out_ref[pl.ds(rf * mc, mc), :nh] = sbuf_f[1].astype(out_ref.dtype)
                out_ref[pl.ds(rb * mc, mc), nh:] = sbuf_b[1].astype(out_ref.dtype)
                src_f = sbuf_f.at[1]
                src_b = sbuf_b.at[1]
            else:
                src_f = rbuf_f.at[(k - 1) % N_DEV]
                src_b = rbuf_b.at[(k - 1) % N_DEV]

            rdma_f = pltpu.make_async_remote_copy(
                src_ref=src_f,
                dst_ref=rbuf_f.at[k % N_DEV],
                send_sem=ssem_f.at[k],
                recv_sem=rsem_f.at[k],
                device_id=(right,),
                device_id_type=pl.DeviceIdType.MESH,
            )
            rdma_b = pltpu.make_async_remote_copy(
                src_ref=src_b,
                dst_ref=rbuf_b.at[k % N_DEV],
                send_sem=ssem_b.at[k],
                recv_sem=rsem_b.at[k],
                device_id=(left,),
                device_id_type=pl.DeviceIdType.MESH,
            )
            rdma_f.start()
            rdma_b.start()
            rdma_f.wait()
            rdma_b.wait()

            if k >= N_DEV - 1:
                t = k - (N_DEV - 1)
                out_ref[pl.ds(((my - t) % N_DEV) * mc, mc), :nh] = rbuf_f[
                    k % N_DEV
                ].astype(out_ref.dtype)
                out_ref[pl.ds(((my + t) % N_DEV) * mc, mc), nh:] = rbuf_b[
                    k % N_DEV
                ].astype(out_ref.dtype)

    nsteps = 2 * (N_DEV - 1)
    return pl.pallas_call(
        body,
        out_shape=jax.ShapeDtypeStruct((m, n), jnp.float32),
        in_specs=[pl.BlockSpec(memory_space=pltpu.VMEM)],
        out_specs=pl.BlockSpec(memory_space=pltpu.VMEM),
        scratch_shapes=[
            pltpu.VMEM((2, mc, nh), jnp.bfloat16),
            pltpu.VMEM((2, mc, nh), jnp.bfloat16),
            pltpu.VMEM((N_DEV, mc, nh), jnp.bfloat16),
            pltpu.VMEM((N_DEV, mc, nh), jnp.bfloat16),
            pltpu.SemaphoreType.DMA((nsteps,)),
            pltpu.SemaphoreType.DMA((nsteps,)),
            pltpu.SemaphoreType.DMA((nsteps,)),
            pltpu.SemaphoreType.DMA((nsteps,)),
        ],
        interpret=pltpu.InterpretParams() if _INTERPRET else False,
    )(x)


# device time: 90821 ns/iter; 2.2405x vs baseline; 1.4349x over previous
import os

import jax
import jax.numpy as jnp
from jax import lax
from jax.experimental import pallas as pl
from jax.experimental.pallas import tpu as pltpu

N_DEV = 8
_INTERPRET = os.environ.get("KERNEL_INTERPRET") == "1"

_MX, _MY, _MZ = 1, 3, 4


def kernel(x):
    m, n = x.shape
    assert m % N_DEV == 0
    mh, mq, me = m // 2, m // 4, m // 8

    w01 = 3 * n // 8
    parts = [
        (0, w01, (_MX, _MY, _MZ)),
        (w01, w01, (_MY, _MZ, _MX)),
        (2 * w01, n - 2 * w01, (_MZ, _MX, _MY)),
    ]

    def body(x_ref, out_ref, acc0, acc1, acc2, rb0, rb1, rb2,
             ss0, rs0, ss1, rs1, ss2, rs2):
        pos = lax.axis_index("i")
        q = lax.bitwise_and(pos, 3)
        ybit = lax.shift_right_logical(q, 1)
        xbit = lax.bitwise_and(lax.bitwise_xor(q, ybit), 1)
        zbit = lax.shift_right_logical(pos, 2)
        bit = {_MX: xbit, _MY: ybit, _MZ: zbit}

        accs = [acc0, acc1, acc2]
        rbs = [rb0, rb1, rb2]
        ssems = [ss0, ss1, ss2]
        rsems = [rs0, rs1, rs2]

        geo = []
        for (c0, w, masks) in parts:
            s0, s1, s2 = (bit[mk] for mk in masks)
            half = s0 * mh
            qtr = half + s1 * mq
            eig = qtr + s2 * me
            geo.append((c0, w, masks, s0, s1, s2, half, qtr, eig))

        def mk(p, step, src, dst, dev):
            return pltpu.make_async_remote_copy(
                src_ref=src,
                dst_ref=dst,
                send_sem=ssems[p].at[step],
                recv_sem=rsems[p].at[step],
                device_id=(dev,),
                device_id_type=pl.DeviceIdType.MESH,
            )

        rdmas = []
        for p, (c0, w, masks, s0, s1, s2, half, qtr, eig) in enumerate(geo):
            dsc = (1 - s0) * mh
            accs[p][pl.ds(dsc, mh), :] = x_ref[pl.ds(dsc, mh), c0:c0 + w].astype(
                jnp.bfloat16
            )
            r = mk(p, 0, accs[p].at[pl.ds(dsc, mh)], rbs[p].at[pl.ds(0, mh)],
                   lax.bitwise_xor(pos, masks[0]))
            r.start()
            rdmas.append(r)
        for p, (c0, w, masks, s0, s1, s2, half, qtr, eig) in enumerate(geo):
            rdmas[p].wait()
            accs[p][pl.ds(half, mh), :] = (
                x_ref[pl.ds(half, mh), c0:c0 + w].astype(jnp.bfloat16)
                + rbs[p][pl.ds(0, mh), :]
            )

        rdmas = []
        for p, (c0, w, masks, s0, s1, s2, half, qtr, eig) in enumerate(geo):
            snd = half + (1 - s1) * mq
            r = mk(p, 1, accs[p].at[pl.ds(snd, mq)], rbs[p].at[pl.ds(mh, mq)],
                   lax.bitwise_xor(pos, masks[1]))
            r.start()
            rdmas.append(r)
        for p, (c0, w, masks, s0, s1, s2, half, qtr, eig) in enumerate(geo):
            rdmas[p].wait()
            accs[p][pl.ds(qtr, mq), :] = (
                accs[p][pl.ds(qtr, mq), :] + rbs[p][pl.ds(mh, mq), :]
            )

        rdmas = []
        for p, (c0, w, masks, s0, s1, s2, half, qtr, eig) in enumerate(geo):
            snd = qtr + (1 - s2) * me
            r = mk(p, 2, accs[p].at[pl.ds(snd, me)],
                   rbs[p].at[pl.ds(mh + mq, me)],
                   lax.bitwise_xor(pos, masks[2]))
            r.start()
            rdmas.append(r)
        for p, (c0, w, masks, s0, s1, s2, half, qtr, eig) in enumerate(geo):
            rdmas[p].wait()
            accs[p][pl.ds(eig, me), :] = (
                accs[p][pl.ds(eig, me), :] + rbs[p][pl.ds(mh + mq, me), :]
            )

        ag = [
            (3, 2, lambda g: g[8], me),
            (4, 1, lambda g: g[7], mq),
            (5, 0, lambda g: g[6], mh),
        ]
        for step, mi, off_fn, size in ag:
            rdmas = []
            for p, g in enumerate(geo):
                off = off_fn(g)
                r = mk(p, step, accs[p].at[pl.ds(off, size)],
                       accs[p].at[pl.ds(off, size)],
                       lax.bitwise_xor(pos, g[2][mi]))
                r.start()
                rdmas.append(r)
            for r in rdmas:
                r.wait()

        for p, (c0, w, *_rest) in enumerate(geo):
            out_ref[:, c0:c0 + w] = accs[p][:, :].astype(out_ref.dtype)

    rb_rows = mh + mq + me
    scratch = []
    for (_c0, w, _masks) in parts:
        scratch.append(pltpu.VMEM((m, w), jnp.bfloat16))
    for (_c0, w, _masks) in parts:
        scratch.append(pltpu.VMEM((rb_rows, w), jnp.bfloat16))
    for _ in parts:
        scratch.append(pltpu.SemaphoreType.DMA((6,)))
        scratch.append(pltpu.SemaphoreType.DMA((6,)))

    return pl.pallas_call(
        body,
        out_shape=jax.ShapeDtypeStruct((m, n), jnp.float32),
        in_specs=[pl.BlockSpec(memory_space=pltpu.VMEM)],
        out_specs=pl.BlockSpec(memory_space=pltpu.VMEM),
        scratch_shapes=scratch,
        interpret=pltpu.InterpretParams() if _INTERPRET else False,
    )(x)


# device time: 86676 ns/iter; 2.3477x vs baseline; 1.0478x over previous
import os

import jax
import jax.numpy as jnp
from jax import lax
from jax.experimental import pallas as pl
from jax.experimental.pallas import tpu as pltpu

N_DEV = 8
_INTERPRET = os.environ.get("KERNEL_INTERPRET") == "1"

_MX, _MY, _MZ = 1, 3, 4


def kernel(x):
    m, n = x.shape
    assert m % N_DEV == 0
    mh, mq, me = m // 2, m // 4, m // 8

    w01 = 3 * n // 8
    parts = [
        (0, w01, (_MX, _MY, _MZ)),
        (w01, w01, (_MY, _MZ, _MX)),
        (2 * w01, n - 2 * w01, (_MZ, _MX, _MY)),
    ]
    wait_order = [2, 0, 1]

    def body(x_ref, out_ref, acc0, acc1, acc2, rb0, rb1, rb2,
             ss0, rs0, ss1, rs1, ss2, rs2):
        pos = lax.axis_index("i")
        q = lax.bitwise_and(pos, 3)
        ybit = lax.shift_right_logical(q, 1)
        xbit = lax.bitwise_and(lax.bitwise_xor(q, ybit), 1)
        zbit = lax.shift_right_logical(pos, 2)
        bit = {_MX: xbit, _MY: ybit, _MZ: zbit}

        accs = [acc0, acc1, acc2]
        rbs = [rb0, rb1, rb2]
        ssems = [ss0, ss1, ss2]
        rsems = [rs0, rs1, rs2]

        geo = []
        for (c0, w, masks) in parts:
            s0, s1, s2 = (bit[mk] for mk in masks)
            half = s0 * mh
            qtr = half + s1 * mq
            eig = qtr + s2 * me
            geo.append((c0, w, masks, s0, s1, s2, half, qtr, eig))

        def mk(p, step, src, dst, mask):
            return pltpu.make_async_remote_copy(
                src_ref=src,
                dst_ref=dst,
                send_sem=ssems[p].at[step],
                recv_sem=rsems[p].at[step],
                device_id=(lax.bitwise_xor(pos, mask),),
                device_id_type=pl.DeviceIdType.MESH,
            )

        rd = {}

        for p, (c0, w, masks, s0, s1, s2, half, qtr, eig) in enumerate(geo):
            dsc = (1 - s0) * mh
            accs[p][pl.ds(dsc, mh), :] = x_ref[pl.ds(dsc, mh), c0:c0 + w].astype(
                jnp.bfloat16
            )
            r = mk(p, 0, accs[p].at[pl.ds(dsc, mh)], rbs[p].at[pl.ds(0, mh)],
                   masks[0])
            r.start()
            rd[p, 0] = r
        for p, (c0, w, masks, s0, s1, s2, half, qtr, eig) in enumerate(geo):
            accs[p][pl.ds(half, mh), :] = x_ref[
                pl.ds(half, mh), c0:c0 + w
            ].astype(jnp.bfloat16)

        for p in wait_order:
            c0, w, masks, s0, s1, s2, half, qtr, eig = geo[p]
            rd[p, 0].wait()
            accs[p][pl.ds(half, mh), :] = (
                accs[p][pl.ds(half, mh), :] + rbs[p][pl.ds(0, mh), :]
            )
            snd = half + (1 - s1) * mq
            r = mk(p, 1, accs[p].at[pl.ds(snd, mq)], rbs[p].at[pl.ds(mh, mq)],
                   masks[1])
            r.start()
            rd[p, 1] = r

        for p in wait_order:
            c0, w, masks, s0, s1, s2, half, qtr, eig = geo[p]
            rd[p, 1].wait()
            accs[p][pl.ds(qtr, mq), :] = (
                accs[p][pl.ds(qtr, mq), :] + rbs[p][pl.ds(mh, mq), :]
            )
            snd = qtr + (1 - s2) * me
            r = mk(p, 2, accs[p].at[pl.ds(snd, me)],
                   rbs[p].at[pl.ds(mh + mq, me)], masks[2])
            r.start()
            rd[p, 2] = r

        for p in wait_order:
            c0, w, masks, s0, s1, s2, half, qtr, eig = geo[p]
            rd[p, 2].wait()
            accs[p][pl.ds(eig, me), :] = (
                accs[p][pl.ds(eig, me), :] + rbs[p][pl.ds(mh + mq, me), :]
            )
            r = mk(p, 3, accs[p].at[pl.ds(eig, me)], accs[p].at[pl.ds(eig, me)],
                   masks[2])
            r.start()
            rd[p, 3] = r

        for p in wait_order:
            c0, w, masks, s0, s1, s2, half, qtr, eig = geo[p]
            rd[p, 3].wait()
            r = mk(p, 4, accs[p].at[pl.ds(qtr, mq)], accs[p].at[pl.ds(qtr, mq)],
                   masks[1])
            r.start()
            rd[p, 4] = r

        for p in wait_order:
            c0, w, masks, s0, s1, s2, half, qtr, eig = geo[p]
            rd[p, 4].wait()
            r = mk(p, 5, accs[p].at[pl.ds(half, mh)], accs[p].at[pl.ds(half, mh)],
                   masks[0])
            r.start()
            rd[p, 5] = r
        for p in wait_order:
            c0, w, masks, s0, s1, s2, half, qtr, eig = geo[p]
            out_ref[pl.ds(half, mh), c0:c0 + w] = accs[p][pl.ds(half, mh), :]
        for p in wait_order:
            c0, w, masks, s0, s1, s2, half, qtr, eig = geo[p]
            rd[p, 5].wait()
            oth = (1 - s0) * mh
            out_ref[pl.ds(oth, mh), c0:c0 + w] = accs[p][pl.ds(oth, mh), :]

    rb_rows = mh + mq + me
    scratch = []
    for (_c0, w, _masks) in parts:
        scratch.append(pltpu.VMEM((m, w), jnp.bfloat16))
    for (_c0, w, _masks) in parts:
        scratch.append(pltpu.VMEM((rb_rows, w), jnp.bfloat16))
    for _ in parts:
        scratch.append(pltpu.SemaphoreType.DMA((6,)))
        scratch.append(pltpu.SemaphoreType.DMA((6,)))

    return pl.pallas_call(
        body,
        out_shape=jax.ShapeDtypeStruct((m, n), jnp.bfloat16),
        in_specs=[pl.BlockSpec(memory_space=pltpu.VMEM)],
        out_specs=pl.BlockSpec(memory_space=pltpu.VMEM),
        scratch_shapes=scratch,
        interpret=pltpu.InterpretParams() if _INTERPRET else False,
    )(x)


# device time: 77196 ns/iter; 2.6360x vs baseline; 1.1228x over previous
import os

import jax
import jax.numpy as jnp
from jax import lax
from jax.experimental import pallas as pl
from jax.experimental.pallas import tpu as pltpu

N_DEV = 8
_INTERPRET = os.environ.get("KERNEL_INTERPRET") == "1"

_MX, _MY, _MZ = 1, 3, 4

_S0A, _S0B, _S1, _S2, _S3, _S4A, _S4B, _S5A, _S5B, _S5C, _S5D = range(11)


def kernel(x):
    m, n = x.shape
    assert m % N_DEV == 0
    mh, mq, me = m // 2, m // 4, m // 8

    w01 = 3 * n // 8
    parts = [
        (0, w01, (_MX, _MY, _MZ)),
        (w01, w01, (_MY, _MZ, _MX)),
        (2 * w01, n - 2 * w01, (_MZ, _MX, _MY)),
    ]
    order = [2, 0, 1]

    def body(x_ref, out_ref, acc0, acc1, acc2, rb0, rb1, rb2,
             ss0, rs0, ss1, rs1, ss2, rs2):
        pos = lax.axis_index("i")
        qq = lax.bitwise_and(pos, 3)
        ybit = lax.shift_right_logical(qq, 1)
        xbit = lax.bitwise_and(lax.bitwise_xor(qq, ybit), 1)
        zbit = lax.shift_right_logical(pos, 2)
        bit = {_MX: xbit, _MY: ybit, _MZ: zbit}

        accs = [acc0, acc1, acc2]
        rbs = [rb0, rb1, rb2]
        ssems = [ss0, ss1, ss2]
        rsems = [rs0, rs1, rs2]

        geo = []
        for (c0, w, masks) in parts:
            s0, s1, s2 = (bit[mk] for mk in masks)
            half = s0 * mh
            dsc = (1 - s0) * mh
            qtr = half + s1 * mq
            eig = qtr + s2 * me
            geo.append(dict(c0=c0, w=w, masks=masks, s0=s0, s1=s1, s2=s2,
                            half=half, dsc=dsc, qtr=qtr, eig=eig))

        def mk(p, slot, src, dst, mask):
            return pltpu.make_async_remote_copy(
                src_ref=src,
                dst_ref=dst,
                send_sem=ssems[p].at[slot],
                recv_sem=rsems[p].at[slot],
                device_id=(lax.bitwise_xor(pos, g_mask[p][mask]),),
                device_id_type=pl.DeviceIdType.MESH,
            )

        g_mask = [g["masks"] for g in geo]
        rd = {}

        for p in order:
            g = geo[p]
            offa = g["dsc"] + (1 - g["s1"]) * mq
            accs[p][pl.ds(offa, mq), :] = x_ref[
                pl.ds(offa, mq), g["c0"]:g["c0"] + g["w"]
            ].astype(jnp.bfloat16)
            r = mk(p, _S0A, accs[p].at[pl.ds(offa, mq)],
                   rbs[p].at[pl.ds(0, mq)], 0)
            r.start()
            rd[p, _S0A] = r
        for p in order:
            g = geo[p]
            offb = g["dsc"] + g["s1"] * mq
            accs[p][pl.ds(offb, mq), :] = x_ref[
                pl.ds(offb, mq), g["c0"]:g["c0"] + g["w"]
            ].astype(jnp.bfloat16)
            r = mk(p, _S0B, accs[p].at[pl.ds(offb, mq)],
                   rbs[p].at[pl.ds(mq, mq)], 0)
            r.start()
            rd[p, _S0B] = r
        for p in order:
            g = geo[p]
            accs[p][pl.ds(g["half"], mh), :] = x_ref[
                pl.ds(g["half"], mh), g["c0"]:g["c0"] + g["w"]
            ].astype(jnp.bfloat16)

        for p in order:
            g = geo[p]
            fwd = g["half"] + (1 - g["s1"]) * mq
            rd[p, _S0A].wait()
            accs[p][pl.ds(fwd, mq), :] = (
                accs[p][pl.ds(fwd, mq), :] + rbs[p][pl.ds(0, mq), :]
            )
            r = mk(p, _S1, accs[p].at[pl.ds(fwd, mq)],
                   rbs[p].at[pl.ds(mh, mq)], 1)
            r.start()
            rd[p, _S1] = r
        for p in order:
            g = geo[p]
            rd[p, _S0B].wait()
            accs[p][pl.ds(g["qtr"], mq), :] = (
                accs[p][pl.ds(g["qtr"], mq), :] + rbs[p][pl.ds(mq, mq), :]
            )
        for p in order:
            g = geo[p]
            rd[p, _S1].wait()
            accs[p][pl.ds(g["qtr"], mq), :] = (
                accs[p][pl.ds(g["qtr"], mq), :] + rbs[p][pl.ds(mh, mq), :]
            )
            snd = g["qtr"] + (1 - g["s2"]) * me
            r = mk(p, _S2, accs[p].at[pl.ds(snd, me)],
                   rbs[p].at[pl.ds(mh + mq, me)], 2)
            r.start()
            rd[p, _S2] = r
        for p in order:
            g = geo[p]
            rd[p, _S2].wait()
            e = g["eig"]
            accs[p][pl.ds(e, me), :] = (
                accs[p][pl.ds(e, me), :] + rbs[p][pl.ds(mh + mq, me), :]
            )
            for slot, mask in ((_S3, 2), (_S4A, 1), (_S5A, 0)):
                r = mk(p, slot, accs[p].at[pl.ds(e, me)],
                       accs[p].at[pl.ds(e, me)], mask)
                r.start()
                rd[p, slot] = r
        for p in order:
            g = geo[p]
            rd[p, _S3].wait()
            o = g["qtr"] + (1 - g["s2"]) * me
            for slot, mask in ((_S4B, 1), (_S5B, 0)):
                r = mk(p, slot, accs[p].at[pl.ds(o, me)],
                       accs[p].at[pl.ds(o, me)], mask)
                r.start()
                rd[p, slot] = r
        for p in order:
            g = geo[p]
            rd[p, _S4A].wait()
            o = g["half"] + (1 - g["s1"]) * mq + g["s2"] * me
            r = mk(p, _S5C, accs[p].at[pl.ds(o, me)],
                   accs[p].at[pl.ds(o, me)], 0)
            r.start()
            rd[p, _S5C] = r
        for p in order:
            g = geo[p]
            rd[p, _S4B].wait()
            o = g["half"] + (1 - g["s1"]) * mq + (1 - g["s2"]) * me
            r = mk(p, _S5D, accs[p].at[pl.ds(o, me)],
                   accs[p].at[pl.ds(o, me)], 0)
            r.start()
            rd[p, _S5D] = r
        for p in order:
            g = geo[p]
            out_ref[pl.ds(g["half"], mh), g["c0"]:g["c0"] + g["w"]] = accs[p][
                pl.ds(g["half"], mh), :
            ]
        for p in order:
            for slot in (_S5A, _S5B, _S5C, _S5D):
                rd[p, slot].wait()
        for p in order:
            g = geo[p]
            out_ref[pl.ds(g["dsc"], mh), g["c0"]:g["c0"] + g["w"]] = accs[p][
                pl.ds(g["dsc"], mh), :
            ]

    rb_rows = mh + mq + me
    scratch = []
    for (_c0, w, _masks) in parts:
        scratch.append(pltpu.VMEM((m, w), jnp.bfloat16))
    for (_c0, w, _masks) in parts:
        scratch.append(pltpu.VMEM((rb_rows, w), jnp.bfloat16))
    for _ in parts:
        scratch.append(pltpu.SemaphoreType.DMA((11,)))
        scratch.append(pltpu.SemaphoreType.DMA((11,)))

    return pl.pallas_call(
        body,
        out_shape=jax.ShapeDtypeStruct((m, n), jnp.bfloat16),
        in_specs=[pl.BlockSpec(memory_space=pltpu.VMEM)],
        out_specs=pl.BlockSpec(memory_space=pltpu.VMEM),
        scratch_shapes=scratch,
        interpret=pltpu.InterpretParams() if _INTERPRET else False,
    )(x)


# device time: 76862 ns/iter; 2.6474x vs baseline; 1.0043x over previous
import os

import jax
import jax.numpy as jnp
from jax import lax
from jax.experimental import pallas as pl
from jax.experimental.pallas import tpu as pltpu

N_DEV = 8
_INTERPRET = os.environ.get("KERNEL_INTERPRET") == "1"

_MX, _MY, _MZ = 1, 3, 4

_S0A, _S0B, _S1, _S2, _S3, _S4A, _S4B, _S5A, _S5B, _S5C, _S5D = range(11)


def kernel(x):
    m, n = x.shape
    assert m % N_DEV == 0
    mh, mq, me = m // 2, m // 4, m // 8

    w01 = 3 * n // 8
    parts = [
        (0, w01, (_MX, _MY, _MZ)),
        (w01, w01, (_MY, _MZ, _MX)),
        (2 * w01, n - 2 * w01, (_MZ, _MX, _MY)),
    ]
    order = [2, 0, 1]

    def body(x_ref, out_ref, st0, st1, st2, rb0, rb1, rb2,
             ss0, rs0, ss1, rs1, ss2, rs2):
        pos = lax.axis_index("i")
        qq = lax.bitwise_and(pos, 3)
        ybit = lax.shift_right_logical(qq, 1)
        xbit = lax.bitwise_and(lax.bitwise_xor(qq, ybit), 1)
        zbit = lax.shift_right_logical(pos, 2)
        bit = {_MX: xbit, _MY: ybit, _MZ: zbit}

        stages = [st0, st1, st2]
        rbs = [rb0, rb1, rb2]
        ssems = [ss0, ss1, ss2]
        rsems = [rs0, rs1, rs2]

        geo = []
        for (c0, w, masks) in parts:
            s0, s1, s2 = (bit[mk] for mk in masks)
            half = s0 * mh
            dsc = (1 - s0) * mh
            qtr = half + s1 * mq
            fwd = half + (1 - s1) * mq
            eig = qtr + s2 * me
            geo.append(dict(c0=c0, w=w, masks=masks, s0=s0, s1=s1, s2=s2,
                            half=half, dsc=dsc, qtr=qtr, fwd=fwd, eig=eig))

        def out_at(p, off, size):
            g = geo[p]
            return out_ref.at[pl.ds(off, size), pl.ds(g["c0"], g["w"])]

        def mk(p, slot, src, dst, dim):
            return pltpu.make_async_remote_copy(
                src_ref=src,
                dst_ref=dst,
                send_sem=ssems[p].at[slot],
                recv_sem=rsems[p].at[slot],
                device_id=(lax.bitwise_xor(pos, geo[p]["masks"][dim]),),
                device_id_type=pl.DeviceIdType.MESH,
            )

        rd = {}

        for p in order:
            g = geo[p]
            offa = g["dsc"] + (1 - g["s1"]) * mq
            stages[p][pl.ds(0, mq), :] = x_ref[
                pl.ds(offa, mq), g["c0"]:g["c0"] + g["w"]
            ].astype(jnp.bfloat16)
            r = mk(p, _S0A, stages[p].at[pl.ds(0, mq)],
                   rbs[p].at[pl.ds(0, mq)], 0)
            r.start()
            rd[p, _S0A] = r
        for p in order:
            g = geo[p]
            offb = g["dsc"] + g["s1"] * mq
            stages[p][pl.ds(mq, mq), :] = x_ref[
                pl.ds(offb, mq), g["c0"]:g["c0"] + g["w"]
            ].astype(jnp.bfloat16)
            r = mk(p, _S0B, stages[p].at[pl.ds(mq, mq)],
                   rbs[p].at[pl.ds(mq, mq)], 0)
            r.start()
            rd[p, _S0B] = r

        for p in order:
            g = geo[p]
            rd[p, _S0A].wait()
            out_ref[pl.ds(g["fwd"], mq), g["c0"]:g["c0"] + g["w"]] = (
                x_ref[pl.ds(g["fwd"], mq), g["c0"]:g["c0"] + g["w"]].astype(
                    jnp.bfloat16
                )
                + rbs[p][pl.ds(0, mq), :]
            )
            r = mk(p, _S1, out_at(p, g["fwd"], mq),
                   rbs[p].at[pl.ds(mh, mq)], 1)
            r.start()
            rd[p, _S1] = r
        for p in order:
            g = geo[p]
            rd[p, _S0B].wait()
            rd[p, _S1].wait()
            out_ref[pl.ds(g["qtr"], mq), g["c0"]:g["c0"] + g["w"]] = (
                x_ref[pl.ds(g["qtr"], mq), g["c0"]:g["c0"] + g["w"]].astype(
                    jnp.bfloat16
                )
                + rbs[p][pl.ds(mq, mq), :]
                + rbs[p][pl.ds(mh, mq), :]
            )
            snd = g["qtr"] + (1 - g["s2"]) * me
            r = mk(p, _S2, out_at(p, snd, me),
                   rbs[p].at[pl.ds(mh + mq, me)], 2)
            r.start()
            rd[p, _S2] = r
        for p in order:
            g = geo[p]
            rd[p, _S2].wait()
            e = g["eig"]
            out_ref[pl.ds(e, me), g["c0"]:g["c0"] + g["w"]] = (
                out_ref[pl.ds(e, me), g["c0"]:g["c0"] + g["w"]]
                + rbs[p][pl.ds(mh + mq, me), :]
            )
            for slot, dim in ((_S3, 2), (_S4A, 1), (_S5A, 0)):
                r = mk(p, slot, out_at(p, e, me), out_at(p, e, me), dim)
                r.start()
                rd[p, slot] = r
        for p in order:
            g = geo[p]
            rd[p, _S3].wait()
            o = g["qtr"] + (1 - g["s2"]) * me
            for slot, dim in ((_S4B, 1), (_S5B, 0)):
                r = mk(p, slot, out_at(p, o, me), out_at(p, o, me), dim)
                r.start()
                rd[p, slot] = r
        for p in order:
            g = geo[p]
            rd[p, _S4A].wait()
            o = g["fwd"] + g["s2"] * me
            r = mk(p, _S5C, out_at(p, o, me), out_at(p, o, me), 0)
            r.start()
            rd[p, _S5C] = r
        for p in order:
            g = geo[p]
            rd[p, _S4B].wait()
            o = g["fwd"] + (1 - g["s2"]) * me
            r = mk(p, _S5D, out_at(p, o, me), out_at(p, o, me), 0)
            r.start()
            rd[p, _S5D] = r
        for p in order:
            for slot in (_S5A, _S5B, _S5C, _S5D):
                rd[p, slot].wait()

    rb_rows = mh + mq + me
    scratch = []
    for (_c0, w, _masks) in parts:
        scratch.append(pltpu.VMEM((mh, w), jnp.bfloat16))
    for (_c0, w, _masks) in parts:
        scratch.append(pltpu.VMEM((rb_rows, w), jnp.bfloat16))
    for _ in parts:
        scratch.append(pltpu.SemaphoreType.DMA((11,)))
        scratch.append(pltpu.SemaphoreType.DMA((11,)))

    return pl.pallas_call(
        body,
        out_shape=jax.ShapeDtypeStruct((m, n), jnp.bfloat16),
        in_specs=[pl.BlockSpec(memory_space=pltpu.VMEM)],
        out_specs=pl.BlockSpec(memory_space=pltpu.VMEM),
        scratch_shapes=scratch,
        interpret=pltpu.InterpretParams() if _INTERPRET else False,
    )(x)


# device time: 72527 ns/iter; 2.8056x vs baseline; 1.0598x over previous
import os

import jax
import jax.numpy as jnp
from jax import lax
from jax.experimental import pallas as pl
from jax.experimental.pallas import tpu as pltpu

N_DEV = 8
_INTERPRET = os.environ.get("KERNEL_INTERPRET") == "1"

_MX, _MY, _MZ = 1, 3, 4

(_S0A, _S0B, _S1A, _S1B, _S2, _S3,
 _S4A, _S4B, _S5A, _S5B, _S5C, _S5D) = range(12)


def kernel(x):
    m, n = x.shape
    assert m % N_DEV == 0
    mh, mq, me = m // 2, m // 4, m // 8

    w01 = 3 * n // 8
    parts = [
        (0, w01, (_MX, _MY, _MZ)),
        (w01, w01, (_MY, _MZ, _MX)),
        (2 * w01, n - 2 * w01, (_MZ, _MX, _MY)),
    ]
    order = [2, 0, 1]

    def body(x_ref, out_ref, st0, st1, st2, rb0, rb1, rb2,
             ss0, rs0, ss1, rs1, ss2, rs2):
        pos = lax.axis_index("i")
        qq = lax.bitwise_and(pos, 3)
        ybit = lax.shift_right_logical(qq, 1)
        xbit = lax.bitwise_and(lax.bitwise_xor(qq, ybit), 1)
        zbit = lax.shift_right_logical(pos, 2)
        bit = {_MX: xbit, _MY: ybit, _MZ: zbit}

        stages = [st0, st1, st2]
        rbs = [rb0, rb1, rb2]
        ssems = [ss0, ss1, ss2]
        rsems = [rs0, rs1, rs2]

        barrier = pltpu.get_barrier_semaphore()
        for mask in (_MX, _MY, _MZ):
            pl.semaphore_signal(
                barrier, inc=1,
                device_id=(lax.bitwise_xor(pos, mask),),
                device_id_type=pl.DeviceIdType.MESH,
            )
        pl.semaphore_wait(barrier, 3)

        geo = []
        for (c0, w, masks) in parts:
            s0, s1, s2 = (bit[mk] for mk in masks)
            half = s0 * mh
            dsc = (1 - s0) * mh
            qtr = half + s1 * mq
            fwd = half + (1 - s1) * mq
            eig = qtr + s2 * me
            geo.append(dict(c0=c0, w=w, masks=masks, s0=s0, s1=s1, s2=s2,
                            half=half, dsc=dsc, qtr=qtr, fwd=fwd, eig=eig))

        def out_at(p, off, size):
            g = geo[p]
            return out_ref.at[pl.ds(off, size), pl.ds(g["c0"], g["w"])]

        def mk(p, slot, src, dst, dim):
            return pltpu.make_async_remote_copy(
                src_ref=src,
                dst_ref=dst,
                send_sem=ssems[p].at[slot],
                recv_sem=rsems[p].at[slot],
                device_id=(lax.bitwise_xor(pos, geo[p]["masks"][dim]),),
                device_id_type=pl.DeviceIdType.MESH,
            )

        rd = {}

        o1a, o1b, o2 = mh, mh + me, mh + mq

        for p in order:
            g = geo[p]
            offa = g["dsc"] + (1 - g["s1"]) * mq
            stages[p][pl.ds(0, mq), :] = x_ref[
                pl.ds(offa, mq), g["c0"]:g["c0"] + g["w"]
            ].astype(jnp.bfloat16)
            r = mk(p, _S0A, stages[p].at[pl.ds(0, mq)],
                   rbs[p].at[pl.ds(0, mq)], 0)
            r.start()
            rd[p, _S0A] = r
        for p in order:
            g = geo[p]
            offb = g["dsc"] + g["s1"] * mq
            stages[p][pl.ds(mq, mq), :] = x_ref[
                pl.ds(offb, mq), g["c0"]:g["c0"] + g["w"]
            ].astype(jnp.bfloat16)
            r = mk(p, _S0B, stages[p].at[pl.ds(mq, mq)],
                   rbs[p].at[pl.ds(mq, mq)], 0)
            r.start()
            rd[p, _S0B] = r

        for p in order:
            g = geo[p]
            rd[p, _S0A].wait()
            out_ref[pl.ds(g["fwd"], mq), g["c0"]:g["c0"] + g["w"]] = (
                x_ref[pl.ds(g["fwd"], mq), g["c0"]:g["c0"] + g["w"]].astype(
                    jnp.bfloat16
                )
                + rbs[p][pl.ds(0, mq), :]
            )
            e1a = g["fwd"] + (1 - g["s2"]) * me
            e1b = g["fwd"] + g["s2"] * me
            r = mk(p, _S1A, out_at(p, e1a, me), rbs[p].at[pl.ds(o1a, me)], 1)
            r.start()
            rd[p, _S1A] = r
            r = mk(p, _S1B, out_at(p, e1b, me), rbs[p].at[pl.ds(o1b, me)], 1)
            r.start()
            rd[p, _S1B] = r
        for p in order:
            g = geo[p]
            rd[p, _S0B].wait()
            rd[p, _S1A].wait()
            o = g["qtr"] + (1 - g["s2"]) * me
            out_ref[pl.ds(o, me), g["c0"]:g["c0"] + g["w"]] = (
                x_ref[pl.ds(o, me), g["c0"]:g["c0"] + g["w"]].astype(
                    jnp.bfloat16
                )
                + rbs[p][pl.ds(mq + (1 - g["s2"]) * me, me), :]
                + rbs[p][pl.ds(o1a, me), :]
            )
            r = mk(p, _S2, out_at(p, o, me), rbs[p].at[pl.ds(o2, me)], 2)
            r.start()
            rd[p, _S2] = r
        for p in order:
            g = geo[p]
            rd[p, _S1B].wait()
            rd[p, _S2].wait()
            e = g["eig"]
            out_ref[pl.ds(e, me), g["c0"]:g["c0"] + g["w"]] = (
                x_ref[pl.ds(e, me), g["c0"]:g["c0"] + g["w"]].astype(
                    jnp.bfloat16
                )
                + rbs[p][pl.ds(mq + g["s2"] * me, me), :]
                + rbs[p][pl.ds(o1b, me), :]
                + rbs[p][pl.ds(o2, me), :]
            )
            for slot, dim in ((_S3, 2), (_S4A, 1), (_S5A, 0)):
                r = mk(p, slot, out_at(p, e, me), out_at(p, e, me), dim)
                r.start()
                rd[p, slot] = r
        for p in order:
            g = geo[p]
            rd[p, _S3].wait()
            o = g["qtr"] + (1 - g["s2"]) * me
            for slot, dim in ((_S4B, 1), (_S5B, 0)):
                r = mk(p, slot, out_at(p, o, me), out_at(p, o, me), dim)
                r.start()
                rd[p, slot] = r
        for p in order:
            g = geo[p]
            rd[p, _S4A].wait()
            o = g["fwd"] + g["s2"] * me
            r = mk(p, _S5C, out_at(p, o, me), out_at(p, o, me), 0)
            r.start()
            rd[p, _S5C] = r
        for p in order:
            g = geo[p]
            rd[p, _S4B].wait()
            o = g["fwd"] + (1 - g["s2"]) * me
            r = mk(p, _S5D, out_at(p, o, me), out_at(p, o, me), 0)
            r.start()
            rd[p, _S5D] = r
        for p in order:
            for slot in (_S5A, _S5B, _S5C, _S5D):
                rd[p, slot].wait()

    rb_rows = mh + mq + me
    scratch = []
    for (_c0, w, _masks) in parts:
        scratch.append(pltpu.VMEM((mh, w), jnp.bfloat16))
    for (_c0, w, _masks) in parts:
        scratch.append(pltpu.VMEM((rb_rows, w), jnp.bfloat16))
    for _ in parts:
        scratch.append(pltpu.SemaphoreType.DMA((12,)))
        scratch.append(pltpu.SemaphoreType.DMA((12,)))

    return pl.pallas_call(
        body,
        out_shape=jax.ShapeDtypeStruct((m, n), jnp.bfloat16),
        in_specs=[pl.BlockSpec(memory_space=pltpu.VMEM)],
        out_specs=pl.BlockSpec(memory_space=pltpu.VMEM),
        scratch_shapes=scratch,
        compiler_params=pltpu.CompilerParams(collective_id=0),
        interpret=pltpu.InterpretParams() if _INTERPRET else False,
    )(x)


# device time: 69090 ns/iter; 2.9452x vs baseline; 1.0497x over previous
import os

import jax
import jax.numpy as jnp
from jax import lax
from jax.experimental import pallas as pl
from jax.experimental.pallas import tpu as pltpu

N_DEV = 8
_INTERPRET = os.environ.get("KERNEL_INTERPRET") == "1"

_MX, _MY, _MZ = 1, 3, 4

(_S0A, _S0B, _S1A, _S1B, _S2, _S3,
 _S4A, _S4B, _S5A, _S5B, _S5C, _S5D) = range(12)


def kernel(x):
    m, n = x.shape
    assert m % N_DEV == 0
    mh, mq, me = m // 2, m // 4, m // 8

    w01 = 3 * n // 8
    parts = [
        (0, w01, (_MX, _MY, _MZ)),
        (w01, w01, (_MY, _MZ, _MX)),
        (2 * w01, n - 2 * w01, (_MZ, _MX, _MY)),
    ]
    order = [2, 0, 1]

    def body(x_ref, out_ref, xv_ref, st0, st1, st2, rb0, rb1, rb2,
             csem, ss0, rs0, ss1, rs1, ss2, rs2):
        pos = lax.axis_index("i")
        qq = lax.bitwise_and(pos, 3)
        ybit = lax.shift_right_logical(qq, 1)
        xbit = lax.bitwise_and(lax.bitwise_xor(qq, ybit), 1)
        zbit = lax.shift_right_logical(pos, 2)
        bit = {_MX: xbit, _MY: ybit, _MZ: zbit}

        stages = [st0, st1, st2]
        rbs = [rb0, rb1, rb2]
        ssems = [ss0, ss1, ss2]
        rsems = [rs0, rs1, rs2]

        barrier = pltpu.get_barrier_semaphore()
        for mask in (_MX, _MY, _MZ):
            pl.semaphore_signal(
                barrier, inc=1,
                device_id=(lax.bitwise_xor(pos, mask),),
                device_id_type=pl.DeviceIdType.MESH,
            )
        pl.semaphore_wait(barrier, 3)

        geo = []
        for (c0, w, masks) in parts:
            s0, s1, s2 = (bit[mk] for mk in masks)
            half = s0 * mh
            dsc = (1 - s0) * mh
            qtr = half + s1 * mq
            fwd = half + (1 - s1) * mq
            eig = qtr + s2 * me
            geo.append(dict(c0=c0, w=w, masks=masks, s0=s0, s1=s1, s2=s2,
                            half=half, dsc=dsc, qtr=qtr, fwd=fwd, eig=eig))

        cps = {}
        for p in order:
            g = geo[p]
            offa = g["dsc"] + (1 - g["s1"]) * mq
            c = pltpu.make_async_copy(
                x_ref.at[pl.ds(offa, mq), pl.ds(g["c0"], g["w"])],
                xv_ref.at[pl.ds(offa, mq), pl.ds(g["c0"], g["w"])],
                csem.at[3 * p + 0],
            )
            c.start()
            cps[p, 0] = c
        for p in order:
            g = geo[p]
            offb = g["dsc"] + g["s1"] * mq
            c = pltpu.make_async_copy(
                x_ref.at[pl.ds(offb, mq), pl.ds(g["c0"], g["w"])],
                xv_ref.at[pl.ds(offb, mq), pl.ds(g["c0"], g["w"])],
                csem.at[3 * p + 1],
            )
            c.start()
            cps[p, 1] = c
        for p in order:
            g = geo[p]
            c = pltpu.make_async_copy(
                x_ref.at[pl.ds(g["half"], mh), pl.ds(g["c0"], g["w"])],
                xv_ref.at[pl.ds(g["half"], mh), pl.ds(g["c0"], g["w"])],
                csem.at[3 * p + 2],
            )
            c.start()
            cps[p, 2] = c

        def out_at(p, off, size):
            g = geo[p]
            return out_ref.at[pl.ds(off, size), pl.ds(g["c0"], g["w"])]

        def mk(p, slot, src, dst, dim):
            return pltpu.make_async_remote_copy(
                src_ref=src,
                dst_ref=dst,
                send_sem=ssems[p].at[slot],
                recv_sem=rsems[p].at[slot],
                device_id=(lax.bitwise_xor(pos, geo[p]["masks"][dim]),),
                device_id_type=pl.DeviceIdType.MESH,
            )

        rd = {}

        o1a, o1b, o2 = mh, mh + me, mh + mq

        for p in order:
            g = geo[p]
            offa = g["dsc"] + (1 - g["s1"]) * mq
            cps[p, 0].wait()
            stages[p][pl.ds(0, mq), :] = xv_ref[
                pl.ds(offa, mq), g["c0"]:g["c0"] + g["w"]
            ].astype(jnp.bfloat16)
            r = mk(p, _S0A, stages[p].at[pl.ds(0, mq)],
                   rbs[p].at[pl.ds(0, mq)], 0)
            r.start()
            rd[p, _S0A] = r
        for p in order:
            g = geo[p]
            offb = g["dsc"] + g["s1"] * mq
            cps[p, 1].wait()
            stages[p][pl.ds(mq, mq), :] = xv_ref[
                pl.ds(offb, mq), g["c0"]:g["c0"] + g["w"]
            ].astype(jnp.bfloat16)
            r = mk(p, _S0B, stages[p].at[pl.ds(mq, mq)],
                   rbs[p].at[pl.ds(mq, mq)], 0)
            r.start()
            rd[p, _S0B] = r

        for p in order:
            g = geo[p]
            rd[p, _S0A].wait()
            cps[p, 2].wait()
            out_ref[pl.ds(g["fwd"], mq), g["c0"]:g["c0"] + g["w"]] = (
                xv_ref[pl.ds(g["fwd"], mq), g["c0"]:g["c0"] + g["w"]].astype(
                    jnp.bfloat16
                )
                + rbs[p][pl.ds(0, mq), :]
            )
            e1a = g["fwd"] + (1 - g["s2"]) * me
            e1b = g["fwd"] + g["s2"] * me
            r = mk(p, _S1A, out_at(p, e1a, me), rbs[p].at[pl.ds(o1a, me)], 1)
            r.start()
            rd[p, _S1A] = r
            r = mk(p, _S1B, out_at(p, e1b, me), rbs[p].at[pl.ds(o1b, me)], 1)
            r.start()
            rd[p, _S1B] = r
        for p in order:
            g = geo[p]
            rd[p, _S0B].wait()
            rd[p, _S1A].wait()
            o = g["qtr"] + (1 - g["s2"]) * me
            out_ref[pl.ds(o, me), g["c0"]:g["c0"] + g["w"]] = (
                xv_ref[pl.ds(o, me), g["c0"]:g["c0"] + g["w"]].astype(
                    jnp.bfloat16
                )
                + rbs[p][pl.ds(mq + (1 - g["s2"]) * me, me), :]
                + rbs[p][pl.ds(o1a, me), :]
            )
            r = mk(p, _S2, out_at(p, o, me), rbs[p].at[pl.ds(o2, me)], 2)
            r.start()
            rd[p, _S2] = r
        for p in order:
            g = geo[p]
            rd[p, _S1B].wait()
            rd[p, _S2].wait()
            e = g["eig"]
            out_ref[pl.ds(e, me), g["c0"]:g["c0"] + g["w"]] = (
                xv_ref[pl.ds(e, me), g["c0"]:g["c0"] + g["w"]].astype(
                    jnp.bfloat16
                )
                + rbs[p][pl.ds(mq + g["s2"] * me, me), :]
                + rbs[p][pl.ds(o1b, me), :]
                + rbs[p][pl.ds(o2, me), :]
            )
            for slot, dim in ((_S3, 2), (_S4A, 1), (_S5A, 0)):
                r = mk(p, slot, out_at(p, e, me), out_at(p, e, me), dim)
                r.start()
                rd[p, slot] = r
        for p in order:
            g = geo[p]
            rd[p, _S3].wait()
            o = g["qtr"] + (1 - g["s2"]) * me
            for slot, dim in ((_S4B, 1), (_S5B, 0)):
                r = mk(p, slot, out_at(p, o, me), out_at(p, o, me), dim)
                r.start()
                rd[p, slot] = r
        for p in order:
            g = geo[p]
            rd[p, _S4A].wait()
            o = g["fwd"] + g["s2"] * me
            r = mk(p, _S5C, out_at(p, o, me), out_at(p, o, me), 0)
            r.start()
            rd[p, _S5C] = r
        for p in order:
            g = geo[p]
            rd[p, _S4B].wait()
            o = g["fwd"] + (1 - g["s2"]) * me
            r = mk(p, _S5D, out_at(p, o, me), out_at(p, o, me), 0)
            r.start()
            rd[p, _S5D] = r
        for p in order:
            for slot in (_S5A, _S5B, _S5C, _S5D):
                rd[p, slot].wait()

    rb_rows = mh + mq + me
    scratch = [pltpu.VMEM((m, n), jnp.float32)]
    for (_c0, w, _masks) in parts:
        scratch.append(pltpu.VMEM((mh, w), jnp.bfloat16))
    for (_c0, w, _masks) in parts:
        scratch.append(pltpu.VMEM((rb_rows, w), jnp.bfloat16))
    scratch.append(pltpu.SemaphoreType.DMA((9,)))
    for _ in parts:
        scratch.append(pltpu.SemaphoreType.DMA((12,)))
        scratch.append(pltpu.SemaphoreType.DMA((12,)))

    return pl.pallas_call(
        body,
        out_shape=jax.ShapeDtypeStruct((m, n), jnp.bfloat16),
        in_specs=[pl.BlockSpec(memory_space=pl.ANY)],
        out_specs=pl.BlockSpec(memory_space=pltpu.VMEM),
        scratch_shapes=scratch,
        compiler_params=pltpu.CompilerParams(collective_id=0),
        interpret=pltpu.InterpretParams() if _INTERPRET else False,
    )(x)


# device time: 68180 ns/iter; 2.9845x vs baseline; 1.0133x over previous
import os

import jax
import jax.numpy as jnp
from jax import lax
from jax.experimental import pallas as pl
from jax.experimental.pallas import tpu as pltpu

N_DEV = 8
_INTERPRET = os.environ.get("KERNEL_INTERPRET") == "1"

_MX, _MY, _MZ = 1, 3, 4

(_S0A, _S0B, _S1A, _S1B, _S2, _S3,
 _S4A, _S4B, _S5A, _S5B, _S5C, _S5D) = range(12)


def kernel(x):
    m, n = x.shape
    assert m % N_DEV == 0
    mh, mq, me = m // 2, m // 4, m // 8

    w01 = 3 * n // 8
    parts = [
        (0, w01, (_MX, _MY, _MZ)),
        (w01, w01, (_MY, _MZ, _MX)),
        (2 * w01, n - 2 * w01, (_MZ, _MX, _MY)),
    ]
    order = [2, 0, 1]
    issue0 = [1, 0, 2]

    def body(x_ref, out_ref, xv_ref, st0, st1, st2, rb0, rb1, rb2,
             csem, ss0, rs0, ss1, rs1, ss2, rs2):
        pos = lax.axis_index("i")
        qq = lax.bitwise_and(pos, 3)
        ybit = lax.shift_right_logical(qq, 1)
        xbit = lax.bitwise_and(lax.bitwise_xor(qq, ybit), 1)
        zbit = lax.shift_right_logical(pos, 2)
        bit = {_MX: xbit, _MY: ybit, _MZ: zbit}

        stages = [st0, st1, st2]
        rbs = [rb0, rb1, rb2]
        ssems = [ss0, ss1, ss2]
        rsems = [rs0, rs1, rs2]

        barrier = pltpu.get_barrier_semaphore()
        for mask in (_MX, _MY, _MZ):
            pl.semaphore_signal(
                barrier, inc=1,
                device_id=(lax.bitwise_xor(pos, mask),),
                device_id_type=pl.DeviceIdType.MESH,
            )
        pl.semaphore_wait(barrier, 3)

        geo = []
        for (c0, w, masks) in parts:
            s0, s1, s2 = (bit[mk] for mk in masks)
            half = s0 * mh
            dsc = (1 - s0) * mh
            qtr = half + s1 * mq
            fwd = half + (1 - s1) * mq
            eig = qtr + s2 * me
            geo.append(dict(c0=c0, w=w, masks=masks, s0=s0, s1=s1, s2=s2,
                            half=half, dsc=dsc, qtr=qtr, fwd=fwd, eig=eig))

        cps = {}
        for p in issue0:
            g = geo[p]
            offa = g["dsc"] + (1 - g["s1"]) * mq
            c = pltpu.make_async_copy(
                x_ref.at[pl.ds(offa, mq), pl.ds(g["c0"], g["w"])],
                xv_ref.at[pl.ds(offa, mq), pl.ds(g["c0"], g["w"])],
                csem.at[3 * p + 0],
            )
            c.start()
            cps[p, 0] = c
        for p in issue0:
            g = geo[p]
            offb = g["dsc"] + g["s1"] * mq
            c = pltpu.make_async_copy(
                x_ref.at[pl.ds(offb, mq), pl.ds(g["c0"], g["w"])],
                xv_ref.at[pl.ds(offb, mq), pl.ds(g["c0"], g["w"])],
                csem.at[3 * p + 1],
            )
            c.start()
            cps[p, 1] = c
        for p in order:
            g = geo[p]
            c = pltpu.make_async_copy(
                x_ref.at[pl.ds(g["half"], mh), pl.ds(g["c0"], g["w"])],
                xv_ref.at[pl.ds(g["half"], mh), pl.ds(g["c0"], g["w"])],
                csem.at[3 * p + 2],
            )
            c.start()
            cps[p, 2] = c

        def out_at(p, off, size):
            g = geo[p]
            return out_ref.at[pl.ds(off, size), pl.ds(g["c0"], g["w"])]

        def mk(p, slot, src, dst, dim):
            return pltpu.make_async_remote_copy(
                src_ref=src,
                dst_ref=dst,
                send_sem=ssems[p].at[slot],
                recv_sem=rsems[p].at[slot],
                device_id=(lax.bitwise_xor(pos, geo[p]["masks"][dim]),),
                device_id_type=pl.DeviceIdType.MESH,
            )

        rd = {}

        o1a, o1b, o2 = mh, mh + me, mh + mq

        for p in issue0:
            g = geo[p]
            offa = g["dsc"] + (1 - g["s1"]) * mq
            cps[p, 0].wait()
            stages[p][pl.ds(0, mq), :] = xv_ref[
                pl.ds(offa, mq), g["c0"]:g["c0"] + g["w"]
            ].astype(jnp.bfloat16)
            r = mk(p, _S0A, stages[p].at[pl.ds(0, mq)],
                   rbs[p].at[pl.ds(0, mq)], 0)
            r.start()
            rd[p, _S0A] = r
        for p in issue0:
            g = geo[p]
            offb = g["dsc"] + g["s1"] * mq
            cps[p, 1].wait()
            stages[p][pl.ds(mq, mq), :] = xv_ref[
                pl.ds(offb, mq), g["c0"]:g["c0"] + g["w"]
            ].astype(jnp.bfloat16)
            r = mk(p, _S0B, stages[p].at[pl.ds(mq, mq)],
                   rbs[p].at[pl.ds(mq, mq)], 0)
            r.start()
            rd[p, _S0B] = r

        for p in order:
            g = geo[p]
            rd[p, _S0A].wait()
            cps[p, 2].wait()
            out_ref[pl.ds(g["fwd"], mq), g["c0"]:g["c0"] + g["w"]] = (
                xv_ref[pl.ds(g["fwd"], mq), g["c0"]:g["c0"] + g["w"]].astype(
                    jnp.bfloat16
                )
                + rbs[p][pl.ds(0, mq), :]
            )
            e1a = g["fwd"] + (1 - g["s2"]) * me
            e1b = g["fwd"] + g["s2"] * me
            r = mk(p, _S1A, out_at(p, e1a, me), rbs[p].at[pl.ds(o1a, me)], 1)
            r.start()
            rd[p, _S1A] = r
            r = mk(p, _S1B, out_at(p, e1b, me), rbs[p].at[pl.ds(o1b, me)], 1)
            r.start()
            rd[p, _S1B] = r
        for p in order:
            g = geo[p]
            rd[p, _S0B].wait()
            rd[p, _S1A].wait()
            o = g["qtr"] + (1 - g["s2"]) * me
            out_ref[pl.ds(o, me), g["c0"]:g["c0"] + g["w"]] = (
                xv_ref[pl.ds(o, me), g["c0"]:g["c0"] + g["w"]].astype(
                    jnp.bfloat16
                )
                + rbs[p][pl.ds(mq + (1 - g["s2"]) * me, me), :]
                + rbs[p][pl.ds(o1a, me), :]
            )
            r = mk(p, _S2, out_at(p, o, me), rbs[p].at[pl.ds(o2, me)], 2)
            r.start()
            rd[p, _S2] = r
        for p in order:
            g = geo[p]
            rd[p, _S1B].wait()
            rd[p, _S2].wait()
            e = g["eig"]
            out_ref[pl.ds(e, me), g["c0"]:g["c0"] + g["w"]] = (
                xv_ref[pl.ds(e, me), g["c0"]:g["c0"] + g["w"]].astype(
                    jnp.bfloat16
                )
                + rbs[p][pl.ds(mq + g["s2"] * me, me), :]
                + rbs[p][pl.ds(o1b, me), :]
                + rbs[p][pl.ds(o2, me), :]
            )
            for slot, dim in ((_S3, 2), (_S4A, 1), (_S5A, 0)):
                r = mk(p, slot, out_at(p, e, me), out_at(p, e, me), dim)
                r.start()
                rd[p, slot] = r
        for p in order:
            g = geo[p]
            rd[p, _S3].wait()
            o = g["qtr"] + (1 - g["s2"]) * me
            for slot, dim in ((_S4B, 1), (_S5B, 0)):
                r = mk(p, slot, out_at(p, o, me), out_at(p, o, me), dim)
                r.start()
                rd[p, slot] = r
        for p in order:
            g = geo[p]
            rd[p, _S4A].wait()
            o = g["fwd"] + g["s2"] * me
            r = mk(p, _S5C, out_at(p, o, me), out_at(p, o, me), 0)
            r.start()
            rd[p, _S5C] = r
        for p in order:
            g = geo[p]
            rd[p, _S4B].wait()
            o = g["fwd"] + (1 - g["s2"]) * me
            r = mk(p, _S5D, out_at(p, o, me), out_at(p, o, me), 0)
            r.start()
            rd[p, _S5D] = r
        for p in order:
            for slot in (_S5A, _S5B, _S5C, _S5D):
                rd[p, slot].wait()

    rb_rows = mh + mq + me
    scratch = [pltpu.VMEM((m, n), jnp.float32)]
    for (_c0, w, _masks) in parts:
        scratch.append(pltpu.VMEM((mh, w), jnp.bfloat16))
    for (_c0, w, _masks) in parts:
        scratch.append(pltpu.VMEM((rb_rows, w), jnp.bfloat16))
    scratch.append(pltpu.SemaphoreType.DMA((9,)))
    for _ in parts:
        scratch.append(pltpu.SemaphoreType.DMA((12,)))
        scratch.append(pltpu.SemaphoreType.DMA((12,)))

    return pl.pallas_call(
        body,
        out_shape=jax.ShapeDtypeStruct((m, n), jnp.bfloat16),
        in_specs=[pl.BlockSpec(memory_space=pl.ANY)],
        out_specs=pl.BlockSpec(memory_space=pltpu.VMEM),
        scratch_shapes=scratch,
        compiler_params=pltpu.CompilerParams(collective_id=0),
        interpret=pltpu.InterpretParams() if _INTERPRET else False,
    )(x)


# device time: 66664 ns/iter; 3.0524x vs baseline; 1.0227x over previous
import os

import jax
import jax.numpy as jnp
from jax import lax
from jax.experimental import pallas as pl
from jax.experimental.pallas import tpu as pltpu

N_DEV = 8
_INTERPRET = os.environ.get("KERNEL_INTERPRET") == "1"

_MX, _MY, _MZ = 1, 3, 4

(_S0A, _S0B, _S1A, _S1B, _S2, _S3,
 _S4A, _S4B, _S5A, _S5B, _S5C, _S5D) = range(12)


def kernel(x):
    m, n = x.shape
    assert m % N_DEV == 0
    mh, mq, me = m // 2, m // 4, m // 8

    w01 = 3 * n // 8
    parts = [
        (0, w01, (_MX, _MY, _MZ)),
        (w01, w01, (_MY, _MZ, _MX)),
        (2 * w01, n - 2 * w01, (_MZ, _MX, _MY)),
    ]
    order = [2, 0, 1]
    issue0 = [1, 0, 2]

    def body(x_ref, out_ref, xv_ref, st0, st1, st2, rb0, rb1, rb2,
             csem, ss0, rs0, ss1, rs1, ss2, rs2):
        pos = lax.axis_index("i")
        qq = lax.bitwise_and(pos, 3)
        ybit = lax.shift_right_logical(qq, 1)
        xbit = lax.bitwise_and(lax.bitwise_xor(qq, ybit), 1)
        zbit = lax.shift_right_logical(pos, 2)
        bit = {_MX: xbit, _MY: ybit, _MZ: zbit}

        stages = [st0, st1, st2]
        rbs = [rb0, rb1, rb2]
        ssems = [ss0, ss1, ss2]
        rsems = [rs0, rs1, rs2]

        geo = []
        for (c0, w, masks) in parts:
            s0, s1, s2 = (bit[mk] for mk in masks)
            half = s0 * mh
            dsc = (1 - s0) * mh
            qtr = half + s1 * mq
            fwd = half + (1 - s1) * mq
            eig = qtr + s2 * me
            geo.append(dict(c0=c0, w=w, masks=masks, s0=s0, s1=s1, s2=s2,
                            half=half, dsc=dsc, qtr=qtr, fwd=fwd, eig=eig))

        cps = {}
        for p in issue0:
            g = geo[p]
            offa = g["dsc"] + (1 - g["s1"]) * mq
            c = pltpu.make_async_copy(
                x_ref.at[pl.ds(offa, mq), pl.ds(g["c0"], g["w"])],
                xv_ref.at[pl.ds(offa, mq), pl.ds(g["c0"], g["w"])],
                csem.at[3 * p + 0],
            )
            c.start()
            cps[p, 0] = c
        for p in issue0:
            g = geo[p]
            offb = g["dsc"] + g["s1"] * mq
            c = pltpu.make_async_copy(
                x_ref.at[pl.ds(offb, mq), pl.ds(g["c0"], g["w"])],
                xv_ref.at[pl.ds(offb, mq), pl.ds(g["c0"], g["w"])],
                csem.at[3 * p + 1],
            )
            c.start()
            cps[p, 1] = c
        for p in order:
            g = geo[p]
            c = pltpu.make_async_copy(
                x_ref.at[pl.ds(g["half"], mh), pl.ds(g["c0"], g["w"])],
                xv_ref.at[pl.ds(g["half"], mh), pl.ds(g["c0"], g["w"])],
                csem.at[3 * p + 2],
            )
            c.start()
            cps[p, 2] = c

        barrier = pltpu.get_barrier_semaphore()
        for mask in (_MX, _MY, _MZ):
            pl.semaphore_signal(
                barrier, inc=1,
                device_id=(lax.bitwise_xor(pos, mask),),
                device_id_type=pl.DeviceIdType.MESH,
            )
        pl.semaphore_wait(barrier, 3)

        def out_at(p, off, size):
            g = geo[p]
            return out_ref.at[pl.ds(off, size), pl.ds(g["c0"], g["w"])]

        def mk(p, slot, src, dst, dim):
            return pltpu.make_async_remote_copy(
                src_ref=src,
                dst_ref=dst,
                send_sem=ssems[p].at[slot],
                recv_sem=rsems[p].at[slot],
                device_id=(lax.bitwise_xor(pos, geo[p]["masks"][dim]),),
                device_id_type=pl.DeviceIdType.MESH,
            )

        rd = {}

        o1a, o1b, o2 = mh, mh + me, mh + mq

        for p in issue0:
            g = geo[p]
            offa = g["dsc"] + (1 - g["s1"]) * mq
            cps[p, 0].wait()
            stages[p][pl.ds(0, mq), :] = xv_ref[
                pl.ds(offa, mq), g["c0"]:g["c0"] + g["w"]
            ].astype(jnp.bfloat16)
            r = mk(p, _S0A, stages[p].at[pl.ds(0, mq)],
                   rbs[p].at[pl.ds(0, mq)], 0)
            r.start()
            rd[p, _S0A] = r
        for p in issue0:
            g = geo[p]
            offb = g["dsc"] + g["s1"] * mq
            cps[p, 1].wait()
            stages[p][pl.ds(mq, mq), :] = xv_ref[
                pl.ds(offb, mq), g["c0"]:g["c0"] + g["w"]
            ].astype(jnp.bfloat16)
            r = mk(p, _S0B, stages[p].at[pl.ds(mq, mq)],
                   rbs[p].at[pl.ds(mq, mq)], 0)
            r.start()
            rd[p, _S0B] = r

        for p in order:
            g = geo[p]
            rd[p, _S0A].wait()
            cps[p, 2].wait()
            out_ref[pl.ds(g["fwd"], mq), g["c0"]:g["c0"] + g["w"]] = (
                xv_ref[pl.ds(g["fwd"], mq), g["c0"]:g["c0"] + g["w"]].astype(
                    jnp.bfloat16
                )
                + rbs[p][pl.ds(0, mq), :]
            )
            e1a = g["fwd"] + (1 - g["s2"]) * me
            e1b = g["fwd"] + g["s2"] * me
            r = mk(p, _S1A, out_at(p, e1a, me), rbs[p].at[pl.ds(o1a, me)], 1)
            r.start()
            rd[p, _S1A] = r
            r = mk(p, _S1B, out_at(p, e1b, me), rbs[p].at[pl.ds(o1b, me)], 1)
            r.start()
            rd[p, _S1B] = r
        for p in order:
            g = geo[p]
            rd[p, _S0B].wait()
            rd[p, _S1A].wait()
            o = g["qtr"] + (1 - g["s2"]) * me
            out_ref[pl.ds(o, me), g["c0"]:g["c0"] + g["w"]] = (
                xv_ref[pl.ds(o, me), g["c0"]:g["c0"] + g["w"]].astype(
                    jnp.bfloat16
                )
                + rbs[p][pl.ds(mq + (1 - g["s2"]) * me, me), :]
                + rbs[p][pl.ds(o1a, me), :]
            )
            r = mk(p, _S2, out_at(p, o, me), rbs[p].at[pl.ds(o2, me)], 2)
            r.start()
            rd[p, _S2] = r
        for p in order:
            g = geo[p]
            rd[p, _S1B].wait()
            rd[p, _S2].wait()
            e = g["eig"]
            out_ref[pl.ds(e, me), g["c0"]:g["c0"] + g["w"]] = (
                xv_ref[pl.ds(e, me), g["c0"]:g["c0"] + g["w"]].astype(
                    jnp.bfloat16
                )
                + rbs[p][pl.ds(mq + g["s2"] * me, me), :]
                + rbs[p][pl.ds(o1b, me), :]
                + rbs[p][pl.ds(o2, me), :]
            )
            for slot, dim in ((_S3, 2), (_S4A, 1), (_S5A, 0)):
                r = mk(p, slot, out_at(p, e, me), out_at(p, e, me), dim)
                r.start()
                rd[p, slot] = r
        for p in order:
            g = geo[p]
            rd[p, _S3].wait()
            o = g["qtr"] + (1 - g["s2"]) * me
            for slot, dim in ((_S4B, 1), (_S5B, 0)):
                r = mk(p, slot, out_at(p, o, me), out_at(p, o, me), dim)
                r.start()
                rd[p, slot] = r
        for p in order:
            g = geo[p]
            rd[p, _S4A].wait()
            o = g["fwd"] + g["s2"] * me
            r = mk(p, _S5C, out_at(p, o, me), out_at(p, o, me), 0)
            r.start()
            rd[p, _S5C] = r
        for p in order:
            g = geo[p]
            rd[p, _S4B].wait()
            o = g["fwd"] + (1 - g["s2"]) * me
            r = mk(p, _S5D, out_at(p, o, me), out_at(p, o, me), 0)
            r.start()
            rd[p, _S5D] = r
        for p in order:
            for slot in (_S5A, _S5B, _S5C, _S5D):
                rd[p, slot].wait()

    rb_rows = mh + mq + me
    scratch = [pltpu.VMEM((m, n), jnp.float32)]
    for (_c0, w, _masks) in parts:
        scratch.append(pltpu.VMEM((mh, w), jnp.bfloat16))
    for (_c0, w, _masks) in parts:
        scratch.append(pltpu.VMEM((rb_rows, w), jnp.bfloat16))
    scratch.append(pltpu.SemaphoreType.DMA((9,)))
    for _ in parts:
        scratch.append(pltpu.SemaphoreType.DMA((12,)))
        scratch.append(pltpu.SemaphoreType.DMA((12,)))

    return pl.pallas_call(
        body,
        out_shape=jax.ShapeDtypeStruct((m, n), jnp.bfloat16),
        in_specs=[pl.BlockSpec(memory_space=pl.ANY)],
        out_specs=pl.BlockSpec(memory_space=pltpu.VMEM),
        scratch_shapes=scratch,
        compiler_params=pltpu.CompilerParams(collective_id=0),
        interpret=pltpu.InterpretParams() if _INTERPRET else False,
    )(x)
